# Initial kernel scaffold; baseline (speedup 1.0000x reference)
import jax
import jax.numpy as jnp
from jax.experimental import pallas as pl


def kernel(x, edge_index, batch, W1, b1, g1, be1, W2, b2, g2, be2, Wout, bout):
    raise NotImplementedError("write your pallas kernel here")


# speedup vs baseline: 12.0580x; 12.0580x over previous
import functools

import jax
import jax.numpy as jnp
from jax import lax
from jax.experimental import pallas as pl
from jax.experimental.pallas import tpu as pltpu
from jax.experimental.pallas import tpu_sc as plsc

N = 10000
E = 320000
D = 128
B = 64

NC = 2
NS = 16
NW = NC * NS
L = 16

NPAD = 10240
RPS = NPAD // NS
EPT = E // NW
K = 80
CH = EPT // K

_mesh = plsc.VectorSubcoreMesh(core_axis_name="c", subcore_axis_name="s")


def _sc_deg(dst):
    @functools.partial(
        pl.kernel, mesh=_mesh,
        out_type=jax.ShapeDtypeStruct((2 * NPAD, 16), jnp.float32),
        scratch_types=[
            pltpu.VMEM((K,), jnp.int32),
            pltpu.VMEM((K, L), jnp.float32),
            pltpu.VMEM((K, L), jnp.float32),
            pltpu.VMEM_SHARED((NPAD, L), jnp.float32),
        ],
    )
    def k(dst_hbm, out_hbm, idx_v, ones_v, zero_v, deg_sh):
        c = lax.axis_index("c")
        s = lax.axis_index("s")
        wid = c * NS + s

        one16 = jnp.ones((L,), jnp.float32)
        zero16 = jnp.zeros((L,), jnp.float32)

        @pl.loop(0, K)
        def _(r):
            ones_v[r] = one16
            zero_v[r] = zero16

        @pl.loop(0, RPS // K)
        def _(j):
            pltpu.sync_copy(zero_v, deg_sh.at[pl.ds(s * RPS + j * K, K)])

        plsc.subcore_barrier()

        base = wid * EPT

        @pl.loop(0, CH)
        def _(g):
            pltpu.sync_copy(dst_hbm.at[pl.ds(base + g * K, K)], idx_v)
            pltpu.sync_copy(ones_v, deg_sh.at[idx_v], add=True)

        plsc.subcore_barrier()
        pltpu.sync_copy(
            deg_sh.at[pl.ds(s * RPS, RPS)],
            out_hbm.at[pl.ds(c * NPAD + s * RPS, RPS)],
        )

    return k(dst)


def _sc_edge(hp, src, dst):
    @functools.partial(
        pl.kernel, mesh=_mesh,
        out_type=jax.ShapeDtypeStruct((2 * NPAD, D), jnp.float32),
        scratch_types=[
            pltpu.VMEM((K,), jnp.int32),
            pltpu.VMEM((K,), jnp.int32),
            pltpu.VMEM((K, D), jnp.float32),
            pltpu.VMEM_SHARED((NPAD, D), jnp.float32),
        ],
    )
    def k(hp_hbm, src_hbm, dst_hbm, out_hbm, si_v, di_v, rows_v, acc_sh):
        c = lax.axis_index("c")
        s = lax.axis_index("s")
        wid = c * NS + s

        zero16 = jnp.zeros((L,), jnp.float32)

        @pl.loop(0, K)
        def _(r):
            @pl.loop(0, D // L)
            def _(j):
                rows_v[r, pl.ds(j * L, L)] = zero16

        @pl.loop(0, RPS // K)
        def _(j):
            pltpu.sync_copy(rows_v, acc_sh.at[pl.ds(s * RPS + j * K, K)])

        plsc.subcore_barrier()

        base = wid * EPT

        @pl.loop(0, CH)
        def _(g):
            pltpu.sync_copy(src_hbm.at[pl.ds(base + g * K, K)], si_v)
            pltpu.sync_copy(dst_hbm.at[pl.ds(base + g * K, K)], di_v)
            pltpu.sync_copy(hp_hbm.at[si_v], rows_v)
            pltpu.sync_copy(rows_v, acc_sh.at[di_v], add=True)

        plsc.subcore_barrier()
        pltpu.sync_copy(
            acc_sh.at[pl.ds(s * RPS, RPS)],
            out_hbm.at[pl.ds(c * NPAD + s * RPS, RPS)],
        )

    return k(hp, src, dst)


R = 1000
NB = N // R


def _tc_mm1(x, deg2, W1):
    def body(x_ref, d_ref, w_ref, h_ref, hp_ref, dis_ref):
        degv = d_ref[0, :, 0:1] + d_ref[1, :, 0:1] + 1.0
        dis = lax.rsqrt(degv)
        h = jnp.dot(x_ref[...], w_ref[...], preferred_element_type=jnp.float32)
        h_ref[...] = h
        hp_ref[...] = h * dis
        dis_ref[...] = dis

    return pl.pallas_call(
        body,
        grid=(NB,),
        in_specs=[
            pl.BlockSpec((R, D), lambda i: (i, 0)),
            pl.BlockSpec((2, R, L), lambda i: (0, i, 0)),
            pl.BlockSpec((D, D), lambda i: (0, 0)),
        ],
        out_specs=[
            pl.BlockSpec((R, D), lambda i: (i, 0)),
            pl.BlockSpec((R, D), lambda i: (i, 0)),
            pl.BlockSpec((R, 1), lambda i: (i, 0)),
        ],
        out_shape=[
            jax.ShapeDtypeStruct((N, D), jnp.float32),
            jax.ShapeDtypeStruct((N, D), jnp.float32),
            jax.ShapeDtypeStruct((N, 1), jnp.float32),
        ],
    )(x, deg2, W1)


def _tc_layer2(acc, h1, x, dis, b1, g1, be1, W2):
    def body(a_ref, h1_ref, x_ref, dis_ref, b1_ref, g1_ref, be1_ref, w2_ref,
             hres_ref, h2_ref, hp2_ref):
        d = dis_ref[...]
        h1 = h1_ref[...]
        o = d * (a_ref[0] + a_ref[1]) + (d * d) * h1 + b1_ref[...]
        m = jnp.mean(o, axis=-1, keepdims=True)
        v = jnp.mean((o - m) ** 2, axis=-1, keepdims=True)
        o = (o - m) * lax.rsqrt(v + 1e-5) * g1_ref[...] + be1_ref[...]
        o = jnp.maximum(o, 0.0) + x_ref[...]
        hres_ref[...] = o
        h2 = jnp.dot(o, w2_ref[...], preferred_element_type=jnp.float32)
        h2_ref[...] = h2
        hp2_ref[...] = h2 * d

    return pl.pallas_call(
        body,
        grid=(NB,),
        in_specs=[
            pl.BlockSpec((2, R, D), lambda i: (0, i, 0)),
            pl.BlockSpec((R, D), lambda i: (i, 0)),
            pl.BlockSpec((R, D), lambda i: (i, 0)),
            pl.BlockSpec((R, 1), lambda i: (i, 0)),
            pl.BlockSpec((1, D), lambda i: (0, 0)),
            pl.BlockSpec((1, D), lambda i: (0, 0)),
            pl.BlockSpec((1, D), lambda i: (0, 0)),
            pl.BlockSpec((D, D), lambda i: (0, 0)),
        ],
        out_specs=[
            pl.BlockSpec((R, D), lambda i: (i, 0)),
            pl.BlockSpec((R, D), lambda i: (i, 0)),
            pl.BlockSpec((R, D), lambda i: (i, 0)),
        ],
        out_shape=[
            jax.ShapeDtypeStruct((N, D), jnp.float32),
            jax.ShapeDtypeStruct((N, D), jnp.float32),
            jax.ShapeDtypeStruct((N, D), jnp.float32),
        ],
    )(acc, h1, x, dis, b1, g1, be1, W2)


def _tc_pool(acc, h2, hres, dis, b2, g2, be2, batch3, Wout, bout):
    def body(a_ref, h2_ref, hres_ref, dis_ref, b2_ref, g2_ref, be2_ref,
             bat_ref, wo_ref, bo_ref, out_ref, sums_ref, cnt_ref):
        i = pl.program_id(0)

        @pl.when(i == 0)
        def _():
            sums_ref[...] = jnp.zeros((B, D), jnp.float32)
            cnt_ref[...] = jnp.zeros((B, 1), jnp.float32)

        d = dis_ref[...]
        h2 = h2_ref[...]
        o = d * (a_ref[0] + a_ref[1]) + (d * d) * h2 + b2_ref[...]
        m = jnp.mean(o, axis=-1, keepdims=True)
        v = jnp.mean((o - m) ** 2, axis=-1, keepdims=True)
        o = (o - m) * lax.rsqrt(v + 1e-5) * g2_ref[...] + be2_ref[...]
        o = jnp.maximum(o, 0.0) + hres_ref[...]

        seg = lax.broadcasted_iota(jnp.int32, (B, 1), 0)
        mask = (bat_ref[0] == seg).astype(jnp.float32)
        sums_ref[...] += jnp.dot(mask, o, preferred_element_type=jnp.float32)
        cnt_ref[...] += jnp.sum(mask, axis=1, keepdims=True)

        @pl.when(i == NB - 1)
        def _():
            hG = sums_ref[...] / jnp.maximum(cnt_ref[...], 1.0)
            out_ref[...] = (
                jnp.dot(hG, wo_ref[...], preferred_element_type=jnp.float32)
                + bo_ref[...]
            )

    return pl.pallas_call(
        body,
        grid=(NB,),
        in_specs=[
            pl.BlockSpec((2, R, D), lambda i: (0, i, 0)),
            pl.BlockSpec((R, D), lambda i: (i, 0)),
            pl.BlockSpec((R, D), lambda i: (i, 0)),
            pl.BlockSpec((R, 1), lambda i: (i, 0)),
            pl.BlockSpec((1, D), lambda i: (0, 0)),
            pl.BlockSpec((1, D), lambda i: (0, 0)),
            pl.BlockSpec((1, D), lambda i: (0, 0)),
            pl.BlockSpec((1, 1, R), lambda i: (i, 0, 0)),
            pl.BlockSpec((D, D), lambda i: (0, 0)),
            pl.BlockSpec((1, D), lambda i: (0, 0)),
        ],
        out_specs=pl.BlockSpec((B, D), lambda i: (0, 0)),
        out_shape=jax.ShapeDtypeStruct((B, D), jnp.float32),
        scratch_shapes=[
            pltpu.VMEM((B, D), jnp.float32),
            pltpu.VMEM((B, 1), jnp.float32),
        ],
    )(acc, h2, hres, dis, b2, g2, be2, batch3, Wout, bout)


def kernel(x, edge_index, batch, W1, b1, g1, be1, W2, b2, g2, be2, Wout, bout):
    src = edge_index[0]
    dst = edge_index[1]
    b1r, g1r, be1r = b1.reshape(1, D), g1.reshape(1, D), be1.reshape(1, D)
    b2r, g2r, be2r = b2.reshape(1, D), g2.reshape(1, D), be2.reshape(1, D)
    boutr = bout.reshape(1, D)
    batch3 = batch.reshape(NB, 1, R)

    deg2 = _sc_deg(dst).reshape(2, NPAD, L)
    h1, hp1, dis = _tc_mm1(x, deg2, W1)
    acc1 = _sc_edge(hp1, src, dst).reshape(2, NPAD, D)
    hres, h2, hp2 = _tc_layer2(acc1, h1, x, dis, b1r, g1r, be1r, W2)
    acc2 = _sc_edge(hp2, src, dst).reshape(2, NPAD, D)
    return _tc_pool(acc2, h2, hres, dis, b2r, g2r, be2r, batch3, Wout, boutr)
